# Initial kernel scaffold; baseline (speedup 1.0000x reference)
import jax
import jax.numpy as jnp
from jax.experimental import pallas as pl


def kernel(x, edge_index, edge_weights, counter_edge, pna_edge_W, pna_edge_b, pna_pre_W, pna_pre_b, pna_post_W, pna_post_b, pna_lin_W, pna_lin_b, g1_Wl, g1_bl, g1_Wr, g1_br, g1_We, g1_att, g1_bias, g2_Wl, g2_bl, g2_Wr, g2_br, g2_We, g2_att, g2_bias, fcn_W, fcn_b, fce1_W, fce1_b, fce2_W, fce2_b):
    raise NotImplementedError("write your pallas kernel here")


# speedup vs baseline: 1.1651x; 1.1651x over previous
import functools

import jax
import jax.numpy as jnp
from jax import lax
from jax.experimental import pallas as pl
from jax.experimental.pallas import tpu as pltpu

N = 100000
E = 1600000

_AVG_LOG = None


def _avg_log():
    global _AVG_LOG
    if _AVG_LOG is None:
        import numpy as np
        hist = np.array([0, 0, 0, 0, 0, 0, 0, 0, 0, 0, 0, 0, 100, 1000, 10000,
                         30000, 40000, 15000, 3000, 800, 100], dtype=np.float32)
        bins = np.arange(len(hist), dtype=np.float32)
        _AVG_LOG = float((np.log(bins + 1.0) * hist).sum() / hist.sum())
    return _AVG_LOG


def _pna_node_block(xb, degb, ssumb, mnb, mxb, msqb, post_W, post_b, lin_W, lin_b, avg_log):
    degc = jnp.maximum(degb[:, 0], 1.0)
    mean = ssumb / degc[:, None]
    mn = jnp.where(jnp.isfinite(mnb), mnb, 0.0)
    mx = jnp.where(jnp.isfinite(mxb), mxb, 0.0)
    msq = msqb / degc[:, None]
    var = msq - mean * mean
    std = jnp.sqrt(jnp.maximum(var, 0.0) + 1e-5)
    agg = jnp.concatenate([mean, mn, mx, std], axis=-1)
    logd = jnp.log(degc + 1.0)[:, None]
    scaled = jnp.concatenate(
        [agg, agg * (logd / avg_log), agg * (avg_log / logd)], axis=-1)
    out = jnp.concatenate([xb, scaled], axis=-1)
    out = out @ post_W + post_b
    return out @ lin_W + lin_b


def _pna_node_kernel(x_ref, deg_ref, ssum_ref, mn_ref, mx_ref, msq_ref,
                     postW_ref, postb_ref, linW_ref, linb_ref, out_ref, *, avg_log):
    out_ref[...] = _pna_node_block(
        x_ref[...], deg_ref[...], ssum_ref[...], mn_ref[...], mx_ref[...],
        msq_ref[...], postW_ref[...], postb_ref[...], linW_ref[...],
        linb_ref[...], avg_log)


def _pna_node(x, deg, ssum, mn, mx, msq, post_W, post_b, lin_W, lin_b):
    B = 2000
    grid = (N // B,)
    deg = deg.reshape(N, 1)
    kfn = functools.partial(_pna_node_kernel, avg_log=_avg_log())
    return pl.pallas_call(
        kfn,
        grid=grid,
        in_specs=[
            pl.BlockSpec((B, 5), lambda i: (i, 0)),
            pl.BlockSpec((B, 1), lambda i: (i, 0)),
            pl.BlockSpec((B, 5), lambda i: (i, 0)),
            pl.BlockSpec((B, 5), lambda i: (i, 0)),
            pl.BlockSpec((B, 5), lambda i: (i, 0)),
            pl.BlockSpec((B, 5), lambda i: (i, 0)),
            pl.BlockSpec((65, 8), lambda i: (0, 0)),
            pl.BlockSpec((8,), lambda i: (0,)),
            pl.BlockSpec((8, 8), lambda i: (0, 0)),
            pl.BlockSpec((8,), lambda i: (0,)),
        ],
        out_specs=pl.BlockSpec((B, 8), lambda i: (i, 0)),
        out_shape=jax.ShapeDtypeStruct((N, 8), jnp.float32),
    )(x, deg, ssum, mn, mx, msq, post_W, post_b, lin_W, lin_b)


def _gat_jnp(x, src, dst, ef, Wl, bl, Wr, br, We, att, bias, H, C, ee_mean_row):
    n = x.shape[0]
    xl = (x @ Wl + bl).reshape(-1, H, C)
    xr = (x @ Wr + br).reshape(-1, H, C)
    ee = (ef @ We).reshape(-1, H, C)
    z = jax.nn.leaky_relu(xl[src] + xr[dst] + ee, 0.2)
    alpha = (z * att[None]).sum(-1)
    ex = jnp.exp(alpha)
    zl = jax.nn.leaky_relu(xl + xr + ee_mean_row.reshape(1, H, C), 0.2)
    al = (zl * att[None]).sum(-1)
    exl = jnp.exp(al)
    denom = jax.ops.segment_sum(ex, dst, num_segments=n) + exl
    numer = jax.ops.segment_sum(xl[src] * ex[:, :, None], dst, num_segments=n)
    numer = numer + xl * exl[:, :, None]
    out = numer / (denom + 1e-16)[:, :, None]
    return out.reshape(n, H * C) + bias


def kernel(x, edge_index, edge_weights, counter_edge, pna_edge_W, pna_edge_b,
           pna_pre_W, pna_pre_b, pna_post_W, pna_post_b, pna_lin_W, pna_lin_b,
           g1_Wl, g1_bl, g1_Wr, g1_br, g1_We, g1_att, g1_bias,
           g2_Wl, g2_bl, g2_Wr, g2_br, g2_We, g2_att, g2_bias,
           fcn_W, fcn_b, fce1_W, fce1_b, fce2_W, fce2_b):
    src = edge_index[0]
    dst = edge_index[1]
    ef = jnp.stack([edge_weights, counter_edge], axis=1)

    Wa = pna_pre_W[0:5]
    Wb = pna_pre_W[5:10]
    Wc = pna_pre_W[10:15]
    Wec = pna_edge_W @ Wc
    b_eff = pna_edge_b @ Wc + pna_pre_b
    u = x @ Wa
    v = x @ Wb
    t = ef @ Wec + b_eff
    m = u[dst] + v[src] + t

    ones = jnp.ones((E,), jnp.float32)
    deg = jax.ops.segment_sum(ones, dst, num_segments=N)
    ssum = jax.ops.segment_sum(m, dst, num_segments=N)
    mn = jax.ops.segment_min(m, dst, num_segments=N)
    mx = jax.ops.segment_max(m, dst, num_segments=N)
    msq = jax.ops.segment_sum(m * m, dst, num_segments=N)

    nf = _pna_node(x, deg, ssum, mn, mx, msq, pna_post_W, pna_post_b,
                   pna_lin_W, pna_lin_b)

    ef_mean = ef.mean(axis=0)
    nf = _gat_jnp(nf, src, dst, ef, g1_Wl, g1_bl, g1_Wr, g1_br, g1_We,
                  g1_att, g1_bias, 4, 4, ef_mean @ g1_We)
    nf = jax.nn.relu(nf)
    nf = _gat_jnp(nf, src, dst, ef, g2_Wl, g2_bl, g2_Wr, g2_br, g2_We,
                  g2_att, g2_bias, 4, 8, ef_mean @ g2_We)
    nf = jax.nn.relu(nf)
    nf = jax.nn.relu(nf @ fcn_W + fcn_b)

    A = fce1_W[0:2] @ fce2_W
    Bm = fce1_W[2:34] @ fce2_W
    Cm = fce1_W[34:66] @ fce2_W
    const = fce1_b @ fce2_W + fce2_b
    ps = nf @ Bm
    pd = nf @ Cm
    out = ef @ A + ps[src] + pd[dst] + const
    return out


# speedup vs baseline: 1.1826x; 1.0151x over previous
import functools

import jax
import jax.numpy as jnp
from jax import lax
from jax.experimental import pallas as pl
from jax.experimental.pallas import tpu as pltpu
from jax.experimental.pallas import tpu_sc as plsc

N = 100000
E = 1600000
NW = 32
EPW = E // NW
L = 16

_AVG_LOG = None


def _avg_log():
    global _AVG_LOG
    if _AVG_LOG is None:
        import numpy as np
        hist = np.array([0, 0, 0, 0, 0, 0, 0, 0, 0, 0, 0, 0, 100, 1000, 10000,
                         30000, 40000, 15000, 3000, 800, 100], dtype=np.float32)
        bins = np.arange(len(hist), dtype=np.float32)
        _AVG_LOG = float((np.log(bins + 1.0) * hist).sum() / hist.sum())
    return _AVG_LOG


def _pna_node_block(xb, degb, ssumb, mnb, mxb, msqb, post_W, post_b, lin_W,
                    lin_b, avg_log):
    degc = jnp.maximum(degb[:, 0], 1.0)
    mean = ssumb / degc[:, None]
    mn = jnp.where(jnp.isfinite(mnb), mnb, 0.0)
    mx = jnp.where(jnp.isfinite(mxb), mxb, 0.0)
    msq = msqb / degc[:, None]
    var = msq - mean * mean
    std = jnp.sqrt(jnp.maximum(var, 0.0) + 1e-5)
    agg = jnp.concatenate([mean, mn, mx, std], axis=-1)
    logd = jnp.log(degc + 1.0)[:, None]
    scaled = jnp.concatenate(
        [agg, agg * (logd / avg_log), agg * (avg_log / logd)], axis=-1)
    out = jnp.concatenate([xb, scaled], axis=-1)
    out = out @ post_W + post_b
    return out @ lin_W + lin_b


def _pna_node_kernel(x_ref, deg_ref, ssum_ref, mn_ref, mx_ref, msq_ref,
                     postW_ref, postb_ref, linW_ref, linb_ref, out_ref, *,
                     avg_log):
    out_ref[...] = _pna_node_block(
        x_ref[...], deg_ref[...], ssum_ref[...], mn_ref[...], mx_ref[...],
        msq_ref[...], postW_ref[...], postb_ref[...], linW_ref[...],
        linb_ref[...], avg_log)


def _pna_node(x, deg, ssum, mn, mx, msq, post_W, post_b, lin_W, lin_b):
    B = 2000
    grid = (N // B,)
    deg = deg.reshape(N, 1)
    kfn = functools.partial(_pna_node_kernel, avg_log=_avg_log())
    return pl.pallas_call(
        kfn,
        grid=grid,
        in_specs=[
            pl.BlockSpec((B, 5), lambda i: (i, 0)),
            pl.BlockSpec((B, 1), lambda i: (i, 0)),
            pl.BlockSpec((B, 5), lambda i: (i, 0)),
            pl.BlockSpec((B, 5), lambda i: (i, 0)),
            pl.BlockSpec((B, 5), lambda i: (i, 0)),
            pl.BlockSpec((B, 5), lambda i: (i, 0)),
            pl.BlockSpec((65, 8), lambda i: (0, 0)),
            pl.BlockSpec((8,), lambda i: (0,)),
            pl.BlockSpec((8, 8), lambda i: (0, 0)),
            pl.BlockSpec((8,), lambda i: (0,)),
        ],
        out_specs=pl.BlockSpec((B, 8), lambda i: (i, 0)),
        out_shape=jax.ShapeDtypeStruct((N, 8), jnp.float32),
    )(x, deg, ssum, mn, mx, msq, post_W, post_b, lin_W, lin_b)


def _edge_out_body(ps0_hbm, ps1_hbm, pd0_hbm, pd1_hbm, src_hbm, dst_hbm,
                   w_hbm, c_hbm, par_hbm, out_hbm,
                   src_v, dst_v, w_v, c_v, g0_v, g1_v, g2_v, g3_v,
                   o0_v, o1_v, par_v, sem, *, K):
    wid = lax.axis_index("s") * 2 + lax.axis_index("c")
    base = wid * EPW
    pltpu.sync_copy(par_hbm, par_v)
    a00 = par_v[0, :]
    a10 = par_v[1, :]
    c0 = par_v[2, :]
    a01 = par_v[3, :]
    a11 = par_v[4, :]
    c1 = par_v[5, :]
    nchunks = EPW // K

    def chunk_body(ci, _):
        cbase = base + ci * K
        pltpu.sync_copy(src_hbm.at[pl.ds(cbase, K)], src_v)
        pltpu.sync_copy(dst_hbm.at[pl.ds(cbase, K)], dst_v)
        pltpu.sync_copy(w_hbm.at[pl.ds(cbase, K)], w_v)
        pltpu.sync_copy(c_hbm.at[pl.ds(cbase, K)], c_v)
        pltpu.async_copy(ps0_hbm.at[src_v], g0_v, sem).wait()
        pltpu.async_copy(ps1_hbm.at[src_v], g1_v, sem).wait()
        pltpu.async_copy(pd0_hbm.at[dst_v], g2_v, sem).wait()
        pltpu.async_copy(pd1_hbm.at[dst_v], g3_v, sem).wait()

        def vec_body(i, _):
            sl = pl.ds(i * L, L)
            w = w_v[sl]
            c = c_v[sl]
            o0_v[sl] = w * a00 + c * a10 + g0_v[sl] + g2_v[sl] + c0
            o1_v[sl] = w * a01 + c * a11 + g1_v[sl] + g3_v[sl] + c1
            return 0

        lax.fori_loop(0, K // L, vec_body, 0)
        pltpu.sync_copy(o0_v, out_hbm.at[pl.ds(cbase, K)])
        pltpu.sync_copy(o1_v, out_hbm.at[pl.ds(E + cbase, K)])
        return 0

    lax.fori_loop(0, nchunks, chunk_body, 0)


def _edge_out_sc(ps0, ps1, pd0, pd1, src, dst, w, c, par):
    K = 2000
    kfn = functools.partial(_edge_out_body, K=K)
    return pl.kernel(
        kfn,
        out_type=jax.ShapeDtypeStruct((2 * E,), jnp.float32),
        name="edge_out_sc",
        mesh=plsc.VectorSubcoreMesh(core_axis_name="c", subcore_axis_name="s"),
        scratch_types=[
            pltpu.VMEM((K,), jnp.int32),
            pltpu.VMEM((K,), jnp.int32),
            pltpu.VMEM((K,), jnp.float32),
            pltpu.VMEM((K,), jnp.float32),
            pltpu.VMEM((K,), jnp.float32),
            pltpu.VMEM((K,), jnp.float32),
            pltpu.VMEM((K,), jnp.float32),
            pltpu.VMEM((K,), jnp.float32),
            pltpu.VMEM((K,), jnp.float32),
            pltpu.VMEM((K,), jnp.float32),
            pltpu.VMEM((8, 16), jnp.float32),
            pltpu.SemaphoreType.DMA,
        ],
    )(ps0, ps1, pd0, pd1, src, dst, w, c, par)


def _gat_layer(nf, src, dst, ef, Wl, bl, Wr, br, We, att, bias, H, C):
    n = N
    xl = (nf @ Wl + bl).reshape(-1, H, C)
    xr = (nf @ Wr + br).reshape(-1, H, C)
    ee = (ef @ We).reshape(-1, H, C)
    z = jax.nn.leaky_relu(xl[src] + xr[dst] + ee, 0.2)
    alpha = (z * att[None]).sum(-1)
    ex = jnp.exp(alpha)
    eem = ef.mean(axis=0) @ We
    zl = jax.nn.leaky_relu(
        (xl.reshape(n, H * C) + xr.reshape(n, H * C)
         + eem[None, :]).reshape(n, H, C), 0.2)
    al = (zl * att[None]).sum(-1)
    exl = jnp.exp(al)
    denom = jax.ops.segment_sum(ex, dst, num_segments=n) + exl
    numer = jax.ops.segment_sum(xl[src] * ex[:, :, None], dst, num_segments=n)
    numer = numer + xl * exl[:, :, None]
    out = numer / (denom + 1e-16)[:, :, None]
    return out.reshape(n, H * C) + bias


def kernel(x, edge_index, edge_weights, counter_edge, pna_edge_W, pna_edge_b,
           pna_pre_W, pna_pre_b, pna_post_W, pna_post_b, pna_lin_W, pna_lin_b,
           g1_Wl, g1_bl, g1_Wr, g1_br, g1_We, g1_att, g1_bias,
           g2_Wl, g2_bl, g2_Wr, g2_br, g2_We, g2_att, g2_bias,
           fcn_W, fcn_b, fce1_W, fce1_b, fce2_W, fce2_b):
    src = edge_index[0]
    dst = edge_index[1]
    ef = jnp.stack([edge_weights, counter_edge], axis=1)

    Wa = pna_pre_W[0:5]
    Wb = pna_pre_W[5:10]
    Wc = pna_pre_W[10:15]
    Wec = pna_edge_W @ Wc
    b_eff = pna_edge_b @ Wc + pna_pre_b
    u = x @ Wa
    v = x @ Wb
    t = ef @ Wec + b_eff
    m = u[dst] + v[src] + t

    ones = jnp.ones((E,), jnp.float32)
    deg = jax.ops.segment_sum(ones, dst, num_segments=N)
    ssum = jax.ops.segment_sum(m, dst, num_segments=N)
    mn = jax.ops.segment_min(m, dst, num_segments=N)
    mx = jax.ops.segment_max(m, dst, num_segments=N)
    msq = jax.ops.segment_sum(m * m, dst, num_segments=N)

    nf = _pna_node(x, deg, ssum, mn, mx, msq, pna_post_W, pna_post_b,
                   pna_lin_W, pna_lin_b)

    nf = _gat_layer(nf, src, dst, ef, g1_Wl, g1_bl, g1_Wr, g1_br, g1_We,
                    g1_att, g1_bias, 4, 4)
    nf = jax.nn.relu(nf)
    nf = _gat_layer(nf, src, dst, ef, g2_Wl, g2_bl, g2_Wr, g2_br, g2_We,
                    g2_att, g2_bias, 4, 8)
    nf = jax.nn.relu(nf)
    nf = jax.nn.relu(nf @ fcn_W + fcn_b)

    A = fce1_W[0:2] @ fce2_W
    Bm = fce1_W[2:34] @ fce2_W
    Cm = fce1_W[34:66] @ fce2_W
    const = fce1_b @ fce2_W + fce2_b
    ps = nf @ Bm
    pd = nf @ Cm
    par = jnp.concatenate([
        A[0, 0:1], A[1, 0:1], const[0:1], A[0, 1:2], A[1, 1:2], const[1:2],
        jnp.zeros((2,), jnp.float32)])
    par = jnp.broadcast_to(par[:, None], (8, 16))
    of = _edge_out_sc(ps[:, 0], ps[:, 1], pd[:, 0], pd[:, 1],
                      src, dst, edge_weights, counter_edge, par)
    return jnp.stack([of[:E], of[E:]], axis=1)
